# Initial kernel scaffold; baseline (speedup 1.0000x reference)
import jax
import jax.numpy as jnp
from jax.experimental import pallas as pl


def kernel(embed_weight, input_ids):
    raise NotImplementedError("write your pallas kernel here")


# speedup vs baseline: 1.6403x; 1.6403x over previous
import functools

import jax
import jax.numpy as jnp
from jax import lax
from jax.experimental import pallas as pl
from jax.experimental.pallas import tpu as pltpu
from jax.experimental.pallas import tpu_sc as plsc

HIDDEN = 3072
NUM_TOKENS = 2 * 4096
NC = 2
NS = 16
NW = NC * NS
PER_W = NUM_TOKENS // NW
CHUNK = 32
NCHUNK = PER_W // CHUNK

_mesh = plsc.VectorSubcoreMesh(core_axis_name="c", subcore_axis_name="s")


@functools.partial(
    pl.kernel,
    mesh=_mesh,
    out_type=jax.ShapeDtypeStruct((NUM_TOKENS, HIDDEN), jnp.float32),
    scratch_types=[
        pltpu.VMEM((NCHUNK, CHUNK), jnp.int32),
        pltpu.VMEM((CHUNK, HIDDEN), jnp.float32),
        pltpu.SemaphoreType.DMA,
    ],
)
def _embed_lookup(table_hbm, ids_hbm, out_hbm, idx_v, rows_v, sem):
    wid = lax.axis_index("s") * NC + lax.axis_index("c")
    base = wid * PER_W
    pltpu.sync_copy(ids_hbm.at[wid], idx_v)
    for j in range(NCHUNK):
        pltpu.async_copy(table_hbm.at[idx_v.at[j]], rows_v, sem).wait()
        pltpu.sync_copy(rows_v, out_hbm.at[pl.ds(base + j * CHUNK, CHUNK)])


def kernel(embed_weight, input_ids):
    batch, seq_len = input_ids.shape
    ids = input_ids.reshape(NW, NCHUNK, CHUNK).astype(jnp.int32)
    out = _embed_lookup(embed_weight, ids)
    return out.reshape(batch, seq_len, HIDDEN)


# speedup vs baseline: 1.6767x; 1.0222x over previous
import functools

import jax
import jax.numpy as jnp
from jax import lax
from jax.experimental import pallas as pl
from jax.experimental.pallas import tpu as pltpu
from jax.experimental.pallas import tpu_sc as plsc

HIDDEN = 3072
NUM_TOKENS = 2 * 4096
NC = 2
NS = 16
NW = NC * NS
PER_W = NUM_TOKENS // NW
CHUNK = 16
NCHUNK = PER_W // CHUNK
NBUF = 2

_mesh = plsc.VectorSubcoreMesh(core_axis_name="c", subcore_axis_name="s")


@functools.partial(
    pl.kernel,
    mesh=_mesh,
    out_type=jax.ShapeDtypeStruct((NUM_TOKENS, HIDDEN), jnp.float32),
    scratch_types=[
        pltpu.VMEM((NCHUNK, CHUNK), jnp.int32),
        pltpu.VMEM((NBUF, CHUNK, HIDDEN), jnp.float32),
        pltpu.SemaphoreType.DMA,
        pltpu.SemaphoreType.DMA,
        pltpu.SemaphoreType.DMA,
        pltpu.SemaphoreType.DMA,
    ],
)
def _embed_lookup(table_hbm, ids_hbm, out_hbm, idx_v, rows_v, si0, si1, so0, so1):
    in_sem = (si0, si1)
    out_sem = (so0, so1)
    wid = lax.axis_index("s") * NC + lax.axis_index("c")
    base = wid * PER_W
    pltpu.sync_copy(ids_hbm.at[wid], idx_v)

    def gather(j, b):
        return pltpu.async_copy(table_hbm.at[idx_v.at[j]], rows_v.at[b], in_sem[b])

    def put(j, b):
        return pltpu.async_copy(
            rows_v.at[b], out_hbm.at[pl.ds(base + j * CHUNK, CHUNK)], out_sem[b]
        )

    gcp = [gather(0, 0), gather(1, 1)]
    pcp = [None, None]
    for j in range(NCHUNK):
        b = j % NBUF
        gcp[b].wait()
        pcp[b] = put(j, b)
        if j + NBUF < NCHUNK:
            pcp[b].wait()
            gcp[b] = gather(j + NBUF, b)
    pcp[0].wait()
    pcp[1].wait()


def kernel(embed_weight, input_ids):
    batch, seq_len = input_ids.shape
    ids = input_ids.reshape(NW, NCHUNK, CHUNK).astype(jnp.int32)
    out = _embed_lookup(embed_weight, ids)
    return out.reshape(batch, seq_len, HIDDEN)


# speedup vs baseline: 1.6860x; 1.0056x over previous
import functools

import jax
import jax.numpy as jnp
from jax import lax
from jax.experimental import pallas as pl
from jax.experimental.pallas import tpu as pltpu
from jax.experimental.pallas import tpu_sc as plsc

HIDDEN = 3072
SEQ = 4096
NUM_TOKENS = 2 * SEQ
NC = 2
NS = 16
NW = NC * NS
PER_W = NUM_TOKENS // NW
CHUNK = 16
NCHUNK = PER_W // CHUNK
NBUF = 2

_mesh = plsc.VectorSubcoreMesh(core_axis_name="c", subcore_axis_name="s")


@functools.partial(
    pl.kernel,
    mesh=_mesh,
    out_type=jax.ShapeDtypeStruct((NUM_TOKENS, HIDDEN), jnp.float32),
    scratch_types=[
        pltpu.VMEM((PER_W,), jnp.int32),
        pltpu.VMEM((NBUF, CHUNK, HIDDEN), jnp.float32),
        pltpu.SemaphoreType.DMA,
        pltpu.SemaphoreType.DMA,
        pltpu.SemaphoreType.DMA,
        pltpu.SemaphoreType.DMA,
    ],
)
def _embed_lookup(table_hbm, ids_hbm, out_hbm, idx_v, rows_v, si0, si1, so0, so1):
    in_sem = (si0, si1)
    out_sem = (so0, so1)
    wid = lax.axis_index("s") * NC + lax.axis_index("c")
    base = wid * PER_W
    tiles_per_row = SEQ // PER_W
    pltpu.sync_copy(
        ids_hbm.at[wid // tiles_per_row, pl.ds((wid % tiles_per_row) * PER_W, PER_W)],
        idx_v,
    )

    def gather(j, b):
        return pltpu.async_copy(
            table_hbm.at[idx_v.at[pl.ds(j * CHUNK, CHUNK)]], rows_v.at[b], in_sem[b]
        )

    def put(j, b):
        return pltpu.async_copy(
            rows_v.at[b], out_hbm.at[pl.ds(base + j * CHUNK, CHUNK)], out_sem[b]
        )

    gcp = [gather(0, 0), gather(1, 1)]
    pcp = [None, None]
    for j in range(NCHUNK):
        b = j % NBUF
        gcp[b].wait()
        pcp[b] = put(j, b)
        if j + NBUF < NCHUNK:
            pcp[b].wait()
            gcp[b] = gather(j + NBUF, b)
    pcp[0].wait()
    pcp[1].wait()


def kernel(embed_weight, input_ids):
    batch, seq_len = input_ids.shape
    out = _embed_lookup(embed_weight, input_ids.astype(jnp.int32))
    return out.reshape(batch, seq_len, HIDDEN)
